# Initial kernel scaffold; baseline (speedup 1.0000x reference)
import jax
import jax.numpy as jnp
from jax.experimental import pallas as pl


def kernel(x, gate_w, gate_b, w1, b1, w2, b2):
    raise NotImplementedError("write your pallas kernel here")


# speedup vs baseline: 6.3050x; 6.3050x over previous
import functools

import jax
import jax.numpy as jnp
from jax import lax
from jax.experimental import pallas as pl
from jax.experimental.pallas import tpu as pltpu
from jax.experimental.pallas import tpu_sc as plsc

D_MODEL = 1024
D_HIDDEN = 4096
N_EXPERT = 8
AUX_COEF = 0.01
N_TOK = 4096
BLK = 256
NB = N_TOK // BLK + N_EXPERT
PADN = NB * BLK
NCHUNK = 32
CHUNK = N_TOK // NCHUNK
HALF = CHUNK // 2
NA_LANE = 120

_SQRT1_2 = 0.7071067811865476


def _gate_kernel(gw_ref, x_ref, gb_ref, dest_ref, be_ref, p_ref, c_ref, aux_ref):
    gw = gw_ref[...]
    x = x_ref[...]
    logits = lax.dot_general(gw, x, (((1,), (1,)), ((), ())),
                             preferred_element_type=jnp.float32)
    logits = logits + gb_ref[...]
    m = jnp.max(logits, axis=0, keepdims=True)
    ex = jnp.exp(logits - m)
    probs = ex / jnp.sum(ex, axis=0, keepdims=True)

    pmax = jnp.max(probs, axis=0, keepdims=True)
    eidx = lax.broadcasted_iota(jnp.int32, (N_EXPERT, N_TOK), 0)
    top1 = jnp.min(jnp.where(probs >= pmax, eidx, N_EXPERT),
                   axis=0, keepdims=True)
    oh = (eidx == top1).astype(jnp.float32)

    counts = jnp.sum(oh, axis=1, keepdims=True)
    p_vec = jnp.sum(probs, axis=1, keepdims=True) / N_TOK
    c_vec = counts / N_TOK
    p_ref[...] = p_vec
    c_ref[...] = c_vec
    aux_ref[...] = jnp.reshape(
        jnp.sum(p_vec * c_vec) * (N_EXPERT * AUX_COEF), (1, 1))

    cnt_i = counts.astype(jnp.int32)
    nb = (cnt_i + (BLK - 1)) // BLK
    e_r = lax.broadcasted_iota(jnp.int32, (N_EXPERT, N_EXPERT), 0)
    e_c = lax.broadcasted_iota(jnp.int32, (N_EXPERT, N_EXPERT), 1)
    l_strict = (e_c < e_r).astype(jnp.float32)
    blk_start = lax.dot_general(l_strict, nb.astype(jnp.float32),
                                (((1,), (0,)), ((), ())),
                                preferred_element_type=jnp.float32)
    blk_start_i = blk_start.astype(jnp.int32)

    bs_b = jnp.broadcast_to(blk_start_i, (N_EXPERT, 128))
    ib = lax.broadcasted_iota(jnp.int32, (N_EXPERT, 128), 1)
    be = jnp.sum((bs_b <= ib).astype(jnp.int32), axis=0, keepdims=True) - 1
    n_active = jnp.sum(nb)
    lane = lax.broadcasted_iota(jnp.int32, (1, 128), 1)
    be_ref[...] = jnp.where(lane == NA_LANE, n_active, be)

    t_i = lax.broadcasted_iota(jnp.int32, (N_TOK, NCHUNK), 0)
    c_i = lax.broadcasted_iota(jnp.int32, (N_TOK, NCHUNK), 1)
    a_mat = ((t_i // CHUNK) == c_i).astype(jnp.float32)
    hist = lax.dot_general(oh, a_mat, (((1,), (0,)), ((), ())),
                           preferred_element_type=jnp.float32)
    cc_r = lax.broadcasted_iota(jnp.int32, (NCHUNK, NCHUNK), 0)
    cc_c = lax.broadcasted_iota(jnp.int32, (NCHUNK, NCHUNK), 1)
    lc = (cc_r < cc_c).astype(jnp.float32)
    cumh = lax.dot_general(hist, lc, (((1,), (0,)), ((), ())),
                           preferred_element_type=jnp.float32)
    pad_off = (blk_start_i * BLK).astype(jnp.float32)
    base = cumh + pad_off

    u_r = lax.broadcasted_iota(jnp.int32, (CHUNK, CHUNK), 0)
    u_c = lax.broadcasted_iota(jnp.int32, (CHUNK, CHUNK), 1)
    t1 = (u_r < u_c).astype(jnp.float32)
    rows = []
    for c in range(NCHUNK):
        ohc = oh[:, CHUNK * c:CHUNK * (c + 1)]
        rank = lax.dot_general(ohc, t1, (((1,), (0,)), ((), ())),
                               preferred_element_type=jnp.float32)
        slot = jnp.sum(ohc * (base[:, c:c + 1] + rank),
                       axis=0, keepdims=True)
        rows.append(slot)
    dest_ref[...] = jnp.concatenate(rows, axis=0).astype(jnp.int32)


def _ffn_kernel(be_ref, x_ref, w1_ref, b1_ref, w2_ref, b2_ref, y_ref):
    i = pl.program_id(0)
    n_active = be_ref[NA_LANE]

    @pl.when(i < n_active)
    def _():
        xb = x_ref[...].astype(jnp.bfloat16)
        w1b = w1_ref[0]
        h = lax.dot_general(xb, w1b, (((1,), (1,)), ((), ())),
                            preferred_element_type=jnp.float32)
        h = h + b1_ref[0]
        h = 0.5 * h * (1.0 + lax.erf(h * _SQRT1_2))
        hb = h.astype(jnp.bfloat16)
        w2b = w2_ref[0]
        y = lax.dot_general(hb, w2b, (((1,), (1,)), ((), ())),
                            preferred_element_type=jnp.float32)
        y_ref[...] = y + b2_ref[0]


@functools.cache
def _sc_kernels():
    mesh = plsc.VectorSubcoreMesh(core_axis_name="c", subcore_axis_name="s")

    @functools.partial(
        pl.kernel, mesh=mesh,
        out_type=jax.ShapeDtypeStruct((PADN, D_MODEL), jnp.float32),
        scratch_types=[
            pltpu.VMEM((2, HALF), jnp.int32),
            pltpu.VMEM((HALF, D_MODEL), jnp.float32),
            pltpu.SemaphoreType.DMA,
        ],
    )
    def sc_scatter(x_hbm, dest_hbm, xs_hbm, didx_v, rows_v, sem):
        wid = lax.axis_index("s") * 2 + lax.axis_index("c")
        pltpu.sync_copy(dest_hbm.at[wid], didx_v)
        for j in range(2):
            base = wid * CHUNK + j * HALF
            pltpu.sync_copy(x_hbm.at[pl.ds(base, HALF)], rows_v)
            pltpu.async_copy(rows_v, xs_hbm.at[didx_v.at[j]], sem).wait()

    @functools.partial(
        pl.kernel, mesh=mesh,
        out_type=jax.ShapeDtypeStruct((N_TOK, D_MODEL), jnp.float32),
        scratch_types=[
            pltpu.VMEM((2, HALF), jnp.int32),
            pltpu.VMEM((HALF, D_MODEL), jnp.float32),
            pltpu.SemaphoreType.DMA,
        ],
    )
    def sc_gather(ys_hbm, dest_hbm, out_hbm, didx_v, rows_v, sem):
        wid = lax.axis_index("s") * 2 + lax.axis_index("c")
        pltpu.sync_copy(dest_hbm.at[wid], didx_v)
        for j in range(2):
            base = wid * CHUNK + j * HALF
            pltpu.async_copy(ys_hbm.at[didx_v.at[j]], rows_v, sem).wait()
            pltpu.sync_copy(rows_v, out_hbm.at[pl.ds(base, HALF)])

    return sc_scatter, sc_gather


def kernel(x, gate_w, gate_b, w1, b1, w2, b2):
    Bb, Ll, D = x.shape
    x_flat = x.reshape(N_TOK, D)

    dest, be, p_col, c_col, aux11 = pl.pallas_call(
        _gate_kernel,
        out_shape=[
            jax.ShapeDtypeStruct((NCHUNK, CHUNK), jnp.int32),
            jax.ShapeDtypeStruct((1, 128), jnp.int32),
            jax.ShapeDtypeStruct((N_EXPERT, 1), jnp.float32),
            jax.ShapeDtypeStruct((N_EXPERT, 1), jnp.float32),
            jax.ShapeDtypeStruct((1, 1), jnp.float32),
        ],
    )(gate_w, x_flat, gate_b.reshape(N_EXPERT, 1))

    sc_scatter, sc_gather = _sc_kernels()
    dest3 = dest.reshape(NCHUNK, 2, HALF)
    x_sorted = sc_scatter(x_flat, dest3)

    be_s = be.reshape(128)
    grid_spec = pltpu.PrefetchScalarGridSpec(
        num_scalar_prefetch=1,
        grid=(NB,),
        in_specs=[
            pl.BlockSpec((BLK, D_MODEL), lambda i, be: (i, 0)),
            pl.BlockSpec((1, D_HIDDEN, D_MODEL), lambda i, be: (be[i], 0, 0)),
            pl.BlockSpec((1, 1, D_HIDDEN), lambda i, be: (be[i], 0, 0)),
            pl.BlockSpec((1, D_MODEL, D_HIDDEN), lambda i, be: (be[i], 0, 0)),
            pl.BlockSpec((1, 1, D_MODEL), lambda i, be: (be[i], 0, 0)),
        ],
        out_specs=pl.BlockSpec((BLK, D_MODEL), lambda i, be: (i, 0)),
    )
    y_sorted = pl.pallas_call(
        _ffn_kernel,
        grid_spec=grid_spec,
        out_shape=jax.ShapeDtypeStruct((PADN, D_MODEL), jnp.float32),
    )(be_s, x_sorted, w1.astype(jnp.bfloat16), b1.reshape(N_EXPERT, 1, D_HIDDEN),
      w2.astype(jnp.bfloat16), b2.reshape(N_EXPERT, 1, D_MODEL))

    out_flat = sc_gather(y_sorted, dest3)

    out = out_flat.reshape(Bb, Ll, D)
    return (out, aux11[0, 0], p_col[:, 0], c_col[:, 0])


# speedup vs baseline: 7.2757x; 1.1539x over previous
import functools

import jax
import jax.numpy as jnp
from jax import lax
from jax.experimental import pallas as pl
from jax.experimental.pallas import tpu as pltpu
from jax.experimental.pallas import tpu_sc as plsc

D_MODEL = 1024
D_HIDDEN = 4096
N_EXPERT = 8
AUX_COEF = 0.01
N_TOK = 4096
BLK = 256
NB = N_TOK // BLK + N_EXPERT
PADN = NB * BLK
NCHUNK = 32
CHUNK = N_TOK // NCHUNK
HALF = CHUNK // 2
NA_LANE = 120

_SQRT1_2 = 0.7071067811865476


def _gate_kernel(gw_ref, x_ref, gb_ref, dest_ref, be_ref, p_ref, c_ref, aux_ref):
    gw = gw_ref[...]
    x = x_ref[...]
    logits = lax.dot_general(gw, x, (((1,), (1,)), ((), ())),
                             preferred_element_type=jnp.float32)
    logits = logits + gb_ref[...]
    m = jnp.max(logits, axis=0, keepdims=True)
    ex = jnp.exp(logits - m)
    probs = ex / jnp.sum(ex, axis=0, keepdims=True)

    pmax = jnp.max(probs, axis=0, keepdims=True)
    eidx = lax.broadcasted_iota(jnp.int32, (N_EXPERT, N_TOK), 0)
    top1 = jnp.min(jnp.where(probs >= pmax, eidx, N_EXPERT),
                   axis=0, keepdims=True)
    oh = (eidx == top1).astype(jnp.float32)

    counts = jnp.sum(oh, axis=1, keepdims=True)
    p_vec = jnp.sum(probs, axis=1, keepdims=True) / N_TOK
    c_vec = counts / N_TOK
    p_ref[...] = p_vec
    c_ref[...] = c_vec
    aux_ref[...] = jnp.reshape(
        jnp.sum(p_vec * c_vec) * (N_EXPERT * AUX_COEF), (1, 1))

    cnt_i = counts.astype(jnp.int32)
    nb = (cnt_i + (BLK - 1)) // BLK
    e_r = lax.broadcasted_iota(jnp.int32, (N_EXPERT, N_EXPERT), 0)
    e_c = lax.broadcasted_iota(jnp.int32, (N_EXPERT, N_EXPERT), 1)
    l_strict = (e_c < e_r).astype(jnp.float32)
    blk_start = lax.dot_general(l_strict, nb.astype(jnp.float32),
                                (((1,), (0,)), ((), ())),
                                preferred_element_type=jnp.float32)
    blk_start_i = blk_start.astype(jnp.int32)

    bs_b = jnp.broadcast_to(blk_start_i, (N_EXPERT, 128))
    ib = lax.broadcasted_iota(jnp.int32, (N_EXPERT, 128), 1)
    be = jnp.sum((bs_b <= ib).astype(jnp.int32), axis=0, keepdims=True) - 1
    n_active = jnp.sum(nb)
    lane = lax.broadcasted_iota(jnp.int32, (1, 128), 1)
    be_ref[...] = jnp.where(lane == NA_LANE, n_active, be)

    t_i = lax.broadcasted_iota(jnp.int32, (N_TOK, NCHUNK), 0)
    c_i = lax.broadcasted_iota(jnp.int32, (N_TOK, NCHUNK), 1)
    a_mat = ((t_i // CHUNK) == c_i).astype(jnp.float32)
    hist = lax.dot_general(oh, a_mat, (((1,), (0,)), ((), ())),
                           preferred_element_type=jnp.float32)
    cc_r = lax.broadcasted_iota(jnp.int32, (NCHUNK, NCHUNK), 0)
    cc_c = lax.broadcasted_iota(jnp.int32, (NCHUNK, NCHUNK), 1)
    lc = (cc_r < cc_c).astype(jnp.float32)
    cumh = lax.dot_general(hist, lc, (((1,), (0,)), ((), ())),
                           preferred_element_type=jnp.float32)
    pad_off = (blk_start_i * BLK).astype(jnp.float32)
    base = cumh + pad_off

    u_r = lax.broadcasted_iota(jnp.int32, (CHUNK, CHUNK), 0)
    u_c = lax.broadcasted_iota(jnp.int32, (CHUNK, CHUNK), 1)
    t1 = (u_r < u_c).astype(jnp.float32)
    rows = []
    for c in range(NCHUNK):
        ohc = oh[:, CHUNK * c:CHUNK * (c + 1)]
        rank = lax.dot_general(ohc, t1, (((1,), (0,)), ((), ())),
                               preferred_element_type=jnp.float32)
        slot = jnp.sum(ohc * (base[:, c:c + 1] + rank),
                       axis=0, keepdims=True)
        rows.append(slot)
    dest_ref[...] = jnp.concatenate(rows, axis=0).astype(jnp.int32)


def _ffn_kernel(be_ref, x_ref, w1_ref, b1_ref, w2_ref, b2_ref, y_ref):
    i = pl.program_id(0)
    n_active = be_ref[NA_LANE]

    @pl.when(i < n_active)
    def _():
        xb = x_ref[...].astype(jnp.bfloat16)
        w1b = w1_ref[0]
        h = lax.dot_general(xb, w1b, (((1,), (1,)), ((), ())),
                            preferred_element_type=jnp.float32)
        h = h + b1_ref[0]
        h = 0.5 * h * (1.0 + lax.erf(h * _SQRT1_2))
        hb = h.astype(jnp.bfloat16)
        w2b = w2_ref[0].astype(jnp.bfloat16)
        y = lax.dot_general(hb, w2b, (((1,), (1,)), ((), ())),
                            preferred_element_type=jnp.float32)
        y_ref[...] = y + b2_ref[0]


@functools.cache
def _sc_kernels():
    mesh = plsc.VectorSubcoreMesh(core_axis_name="c", subcore_axis_name="s")

    @functools.partial(
        pl.kernel, mesh=mesh,
        out_type=jax.ShapeDtypeStruct((PADN, D_MODEL), jnp.float32),
        scratch_types=[
            pltpu.VMEM((2, HALF), jnp.int32),
            pltpu.VMEM((HALF, D_MODEL), jnp.float32),
            pltpu.SemaphoreType.DMA,
        ],
    )
    def sc_scatter(x_hbm, dest_hbm, xs_hbm, didx_v, rows_v, sem):
        wid = lax.axis_index("s") * 2 + lax.axis_index("c")
        pltpu.sync_copy(dest_hbm.at[wid], didx_v)
        for j in range(2):
            base = wid * CHUNK + j * HALF
            pltpu.sync_copy(x_hbm.at[pl.ds(base, HALF)], rows_v)
            pltpu.async_copy(rows_v, xs_hbm.at[didx_v.at[j]], sem).wait()

    @functools.partial(
        pl.kernel, mesh=mesh,
        out_type=jax.ShapeDtypeStruct((N_TOK, D_MODEL), jnp.float32),
        scratch_types=[
            pltpu.VMEM((2, HALF), jnp.int32),
            pltpu.VMEM((HALF, D_MODEL), jnp.float32),
            pltpu.SemaphoreType.DMA,
        ],
    )
    def sc_gather(ys_hbm, dest_hbm, out_hbm, didx_v, rows_v, sem):
        wid = lax.axis_index("s") * 2 + lax.axis_index("c")
        pltpu.sync_copy(dest_hbm.at[wid], didx_v)
        for j in range(2):
            base = wid * CHUNK + j * HALF
            pltpu.async_copy(ys_hbm.at[didx_v.at[j]], rows_v, sem).wait()
            pltpu.sync_copy(rows_v, out_hbm.at[pl.ds(base, HALF)])

    return sc_scatter, sc_gather


def kernel(x, gate_w, gate_b, w1, b1, w2, b2):
    Bb, Ll, D = x.shape
    x_flat = x.reshape(N_TOK, D)

    dest, be, p_col, c_col, aux11 = pl.pallas_call(
        _gate_kernel,
        out_shape=[
            jax.ShapeDtypeStruct((NCHUNK, CHUNK), jnp.int32),
            jax.ShapeDtypeStruct((1, 128), jnp.int32),
            jax.ShapeDtypeStruct((N_EXPERT, 1), jnp.float32),
            jax.ShapeDtypeStruct((N_EXPERT, 1), jnp.float32),
            jax.ShapeDtypeStruct((1, 1), jnp.float32),
        ],
    )(gate_w, x_flat, gate_b.reshape(N_EXPERT, 1))

    sc_scatter, sc_gather = _sc_kernels()
    dest3 = dest.reshape(NCHUNK, 2, HALF)
    x_sorted = sc_scatter(x_flat, dest3)

    be_s = be.reshape(128)
    grid_spec = pltpu.PrefetchScalarGridSpec(
        num_scalar_prefetch=1,
        grid=(NB,),
        in_specs=[
            pl.BlockSpec((BLK, D_MODEL), lambda i, be: (i, 0)),
            pl.BlockSpec((1, D_HIDDEN, D_MODEL), lambda i, be: (be[i], 0, 0)),
            pl.BlockSpec((1, 1, D_HIDDEN), lambda i, be: (be[i], 0, 0)),
            pl.BlockSpec((1, D_MODEL, D_HIDDEN), lambda i, be: (be[i], 0, 0)),
            pl.BlockSpec((1, 1, D_MODEL), lambda i, be: (be[i], 0, 0)),
        ],
        out_specs=pl.BlockSpec((BLK, D_MODEL), lambda i, be: (i, 0)),
    )
    y_sorted = pl.pallas_call(
        _ffn_kernel,
        grid_spec=grid_spec,
        out_shape=jax.ShapeDtypeStruct((PADN, D_MODEL), jnp.float32),
        compiler_params=pltpu.CompilerParams(
            vmem_limit_bytes=112 * 1024 * 1024),
    )(be_s, x_sorted, w1.astype(jnp.bfloat16), b1.reshape(N_EXPERT, 1, D_HIDDEN),
      w2, b2.reshape(N_EXPERT, 1, D_MODEL))

    out_flat = sc_gather(y_sorted, dest3)

    out = out_flat.reshape(Bb, Ll, D)
    return (out, aux11[0, 0], p_col[:, 0], c_col[:, 0])


# speedup vs baseline: 7.2872x; 1.0016x over previous
import functools

import jax
import jax.numpy as jnp
from jax import lax
from jax.experimental import pallas as pl
from jax.experimental.pallas import tpu as pltpu
from jax.experimental.pallas import tpu_sc as plsc

D_MODEL = 1024
D_HIDDEN = 4096
N_EXPERT = 8
AUX_COEF = 0.01
N_TOK = 4096
BLK = 256
NB = N_TOK // BLK + N_EXPERT
PADN = NB * BLK
NCHUNK = 32
CHUNK = N_TOK // NCHUNK
HALF = CHUNK // 2
NA_LANE = 120

_SQRT1_2 = 0.7071067811865476


def _gate_kernel(gw_ref, x_ref, gb_ref, dest_ref, be_ref, p_ref, c_ref, aux_ref):
    gw = gw_ref[...]
    x = x_ref[...]
    logits = lax.dot_general(gw, x, (((1,), (1,)), ((), ())),
                             preferred_element_type=jnp.float32)
    logits = logits + gb_ref[...]
    m = jnp.max(logits, axis=0, keepdims=True)
    ex = jnp.exp(logits - m)
    probs = ex / jnp.sum(ex, axis=0, keepdims=True)

    pmax = jnp.max(probs, axis=0, keepdims=True)
    eidx = lax.broadcasted_iota(jnp.int32, (N_EXPERT, N_TOK), 0)
    top1 = jnp.min(jnp.where(probs >= pmax, eidx, N_EXPERT),
                   axis=0, keepdims=True)
    oh = (eidx == top1).astype(jnp.float32)

    counts = jnp.sum(oh, axis=1, keepdims=True)
    p_vec = jnp.sum(probs, axis=1, keepdims=True) / N_TOK
    c_vec = counts / N_TOK
    p_ref[...] = p_vec
    c_ref[...] = c_vec
    aux_ref[...] = jnp.reshape(
        jnp.sum(p_vec * c_vec) * (N_EXPERT * AUX_COEF), (1, 1))

    cnt_i = counts.astype(jnp.int32)
    nb = (cnt_i + (BLK - 1)) // BLK
    e_r = lax.broadcasted_iota(jnp.int32, (N_EXPERT, N_EXPERT), 0)
    e_c = lax.broadcasted_iota(jnp.int32, (N_EXPERT, N_EXPERT), 1)
    l_strict = (e_c < e_r).astype(jnp.float32)
    blk_start = lax.dot_general(l_strict, nb.astype(jnp.float32),
                                (((1,), (0,)), ((), ())),
                                preferred_element_type=jnp.float32)
    blk_start_i = blk_start.astype(jnp.int32)

    bs_b = jnp.broadcast_to(blk_start_i, (N_EXPERT, 128))
    ib = lax.broadcasted_iota(jnp.int32, (N_EXPERT, 128), 1)
    be = jnp.sum((bs_b <= ib).astype(jnp.int32), axis=0, keepdims=True) - 1
    n_active = jnp.sum(nb)
    lane = lax.broadcasted_iota(jnp.int32, (1, 128), 1)
    be_ref[...] = jnp.where(lane == NA_LANE, n_active, be)

    t_i = lax.broadcasted_iota(jnp.int32, (N_TOK, NCHUNK), 0)
    c_i = lax.broadcasted_iota(jnp.int32, (N_TOK, NCHUNK), 1)
    a_mat = ((t_i // CHUNK) == c_i).astype(jnp.float32)
    hist = lax.dot_general(oh, a_mat, (((1,), (0,)), ((), ())),
                           preferred_element_type=jnp.float32)
    cc_r = lax.broadcasted_iota(jnp.int32, (NCHUNK, NCHUNK), 0)
    cc_c = lax.broadcasted_iota(jnp.int32, (NCHUNK, NCHUNK), 1)
    lc = (cc_r < cc_c).astype(jnp.float32)
    cumh = lax.dot_general(hist, lc, (((1,), (0,)), ((), ())),
                           preferred_element_type=jnp.float32)
    pad_off = (blk_start_i * BLK).astype(jnp.float32)
    base = cumh + pad_off

    u_r = lax.broadcasted_iota(jnp.int32, (CHUNK, CHUNK), 0)
    u_c = lax.broadcasted_iota(jnp.int32, (CHUNK, CHUNK), 1)
    t1 = (u_r < u_c).astype(jnp.float32)
    rows = []
    for c in range(NCHUNK):
        ohc = oh[:, CHUNK * c:CHUNK * (c + 1)]
        rank = lax.dot_general(ohc, t1, (((1,), (0,)), ((), ())),
                               preferred_element_type=jnp.float32)
        slot = jnp.sum(ohc * (base[:, c:c + 1] + rank),
                       axis=0, keepdims=True)
        rows.append(slot)
    dest_ref[...] = jnp.concatenate(rows, axis=0).astype(jnp.int32)


def _ffn1_kernel(be_ref, x_ref, w1_ref, b1_ref, h_ref):
    i = pl.program_id(0)
    n_active = be_ref[NA_LANE]

    @pl.when(i < n_active)
    def _():
        xb = x_ref[...].astype(jnp.bfloat16)
        w1b = w1_ref[0].astype(jnp.bfloat16)
        h = lax.dot_general(xb, w1b, (((1,), (1,)), ((), ())),
                            preferred_element_type=jnp.float32)
        h = h + b1_ref[0]
        h = 0.5 * h * (1.0 + lax.erf(h * _SQRT1_2))
        h_ref[...] = h.astype(jnp.bfloat16)


def _ffn2_kernel(be_ref, h_ref, w2_ref, b2_ref, y_ref):
    i = pl.program_id(0)
    n_active = be_ref[NA_LANE]

    @pl.when(i < n_active)
    def _():
        w2b = w2_ref[0].astype(jnp.bfloat16)
        y = lax.dot_general(h_ref[...], w2b, (((1,), (1,)), ((), ())),
                            preferred_element_type=jnp.float32)
        y_ref[...] = y + b2_ref[0]


@functools.cache
def _sc_kernels():
    mesh = plsc.VectorSubcoreMesh(core_axis_name="c", subcore_axis_name="s")

    @functools.partial(
        pl.kernel, mesh=mesh,
        out_type=jax.ShapeDtypeStruct((PADN, D_MODEL), jnp.float32),
        scratch_types=[
            pltpu.VMEM((2, HALF), jnp.int32),
            pltpu.VMEM((HALF, D_MODEL), jnp.float32),
            pltpu.SemaphoreType.DMA,
        ],
    )
    def sc_scatter(x_hbm, dest_hbm, xs_hbm, didx_v, rows_v, sem):
        wid = lax.axis_index("s") * 2 + lax.axis_index("c")
        pltpu.sync_copy(dest_hbm.at[wid], didx_v)
        for j in range(2):
            base = wid * CHUNK + j * HALF
            pltpu.sync_copy(x_hbm.at[pl.ds(base, HALF)], rows_v)
            pltpu.async_copy(rows_v, xs_hbm.at[didx_v.at[j]], sem).wait()

    @functools.partial(
        pl.kernel, mesh=mesh,
        out_type=jax.ShapeDtypeStruct((N_TOK, D_MODEL), jnp.float32),
        scratch_types=[
            pltpu.VMEM((2, HALF), jnp.int32),
            pltpu.VMEM((HALF, D_MODEL), jnp.float32),
            pltpu.SemaphoreType.DMA,
        ],
    )
    def sc_gather(ys_hbm, dest_hbm, out_hbm, didx_v, rows_v, sem):
        wid = lax.axis_index("s") * 2 + lax.axis_index("c")
        pltpu.sync_copy(dest_hbm.at[wid], didx_v)
        for j in range(2):
            base = wid * CHUNK + j * HALF
            pltpu.async_copy(ys_hbm.at[didx_v.at[j]], rows_v, sem).wait()
            pltpu.sync_copy(rows_v, out_hbm.at[pl.ds(base, HALF)])

    return sc_scatter, sc_gather


def kernel(x, gate_w, gate_b, w1, b1, w2, b2):
    Bb, Ll, D = x.shape
    x_flat = x.reshape(N_TOK, D)

    dest, be, p_col, c_col, aux11 = pl.pallas_call(
        _gate_kernel,
        out_shape=[
            jax.ShapeDtypeStruct((NCHUNK, CHUNK), jnp.int32),
            jax.ShapeDtypeStruct((1, 128), jnp.int32),
            jax.ShapeDtypeStruct((N_EXPERT, 1), jnp.float32),
            jax.ShapeDtypeStruct((N_EXPERT, 1), jnp.float32),
            jax.ShapeDtypeStruct((1, 1), jnp.float32),
        ],
    )(gate_w, x_flat, gate_b.reshape(N_EXPERT, 1))

    sc_scatter, sc_gather = _sc_kernels()
    dest3 = dest.reshape(NCHUNK, 2, HALF)
    x_sorted = sc_scatter(x_flat, dest3)

    be_s = be.reshape(128)
    h_sorted = pl.pallas_call(
        _ffn1_kernel,
        grid_spec=pltpu.PrefetchScalarGridSpec(
            num_scalar_prefetch=1,
            grid=(NB,),
            in_specs=[
                pl.BlockSpec((BLK, D_MODEL), lambda i, be: (i, 0)),
                pl.BlockSpec((1, D_HIDDEN, D_MODEL), lambda i, be: (be[i], 0, 0)),
                pl.BlockSpec((1, 1, D_HIDDEN), lambda i, be: (be[i], 0, 0)),
            ],
            out_specs=pl.BlockSpec((BLK, D_HIDDEN), lambda i, be: (i, 0)),
        ),
        out_shape=jax.ShapeDtypeStruct((PADN, D_HIDDEN), jnp.bfloat16),
        compiler_params=pltpu.CompilerParams(
            vmem_limit_bytes=60 * 1024 * 1024),
    )(be_s, x_sorted, w1, b1.reshape(N_EXPERT, 1, D_HIDDEN))

    y_sorted = pl.pallas_call(
        _ffn2_kernel,
        grid_spec=pltpu.PrefetchScalarGridSpec(
            num_scalar_prefetch=1,
            grid=(NB,),
            in_specs=[
                pl.BlockSpec((BLK, D_HIDDEN), lambda i, be: (i, 0)),
                pl.BlockSpec((1, D_MODEL, D_HIDDEN), lambda i, be: (be[i], 0, 0)),
                pl.BlockSpec((1, 1, D_MODEL), lambda i, be: (be[i], 0, 0)),
            ],
            out_specs=pl.BlockSpec((BLK, D_MODEL), lambda i, be: (i, 0)),
        ),
        out_shape=jax.ShapeDtypeStruct((PADN, D_MODEL), jnp.float32),
        compiler_params=pltpu.CompilerParams(
            vmem_limit_bytes=60 * 1024 * 1024),
    )(be_s, h_sorted, w2, b2.reshape(N_EXPERT, 1, D_MODEL))

    out_flat = sc_gather(y_sorted, dest3)

    out = out_flat.reshape(Bb, Ll, D)
    return (out, aux11[0, 0], p_col[:, 0], c_col[:, 0])


# speedup vs baseline: 7.5551x; 1.0368x over previous
import functools

import jax
import jax.numpy as jnp
from jax import lax
from jax.experimental import pallas as pl
from jax.experimental.pallas import tpu as pltpu
from jax.experimental.pallas import tpu_sc as plsc

D_MODEL = 1024
D_HIDDEN = 4096
N_EXPERT = 8
AUX_COEF = 0.01
N_TOK = 4096
BLK = 256
NB = N_TOK // BLK + N_EXPERT
PADN = NB * BLK
NCHUNK = 32
CHUNK = N_TOK // NCHUNK
NLEG = 4
LEG = CHUNK // NLEG
NA_LANE = 120

_SQRT1_2 = 0.7071067811865476


def _gate_kernel(gw_ref, x_ref, gb_ref, dest_ref, be_ref, p_ref, c_ref, aux_ref):
    gw = gw_ref[...]
    x = x_ref[...]
    logits = lax.dot_general(gw, x, (((1,), (1,)), ((), ())),
                             preferred_element_type=jnp.float32)
    logits = logits + gb_ref[...]
    m = jnp.max(logits, axis=0, keepdims=True)
    ex = jnp.exp(logits - m)
    probs = ex / jnp.sum(ex, axis=0, keepdims=True)

    pmax = jnp.max(probs, axis=0, keepdims=True)
    eidx = lax.broadcasted_iota(jnp.int32, (N_EXPERT, N_TOK), 0)
    top1 = jnp.min(jnp.where(probs >= pmax, eidx, N_EXPERT),
                   axis=0, keepdims=True)
    oh = (eidx == top1).astype(jnp.float32)

    counts = jnp.sum(oh, axis=1, keepdims=True)
    p_vec = jnp.sum(probs, axis=1, keepdims=True) / N_TOK
    c_vec = counts / N_TOK
    p_ref[...] = p_vec
    c_ref[...] = c_vec
    aux_ref[...] = jnp.reshape(
        jnp.sum(p_vec * c_vec) * (N_EXPERT * AUX_COEF), (1, 1))

    cnt_i = counts.astype(jnp.int32)
    nb = (cnt_i + (BLK - 1)) // BLK
    e_r = lax.broadcasted_iota(jnp.int32, (N_EXPERT, N_EXPERT), 0)
    e_c = lax.broadcasted_iota(jnp.int32, (N_EXPERT, N_EXPERT), 1)
    l_strict = (e_c < e_r).astype(jnp.float32)
    blk_start = lax.dot_general(l_strict, nb.astype(jnp.float32),
                                (((1,), (0,)), ((), ())),
                                preferred_element_type=jnp.float32)
    blk_start_i = blk_start.astype(jnp.int32)

    bs_b = jnp.broadcast_to(blk_start_i, (N_EXPERT, 128))
    ib = lax.broadcasted_iota(jnp.int32, (N_EXPERT, 128), 1)
    be = jnp.sum((bs_b <= ib).astype(jnp.int32), axis=0, keepdims=True) - 1
    n_active = jnp.sum(nb)
    lane = lax.broadcasted_iota(jnp.int32, (1, 128), 1)
    be_ref[...] = jnp.where(lane == NA_LANE, n_active, be)

    t_i = lax.broadcasted_iota(jnp.int32, (N_TOK, NCHUNK), 0)
    c_i = lax.broadcasted_iota(jnp.int32, (N_TOK, NCHUNK), 1)
    a_mat = ((t_i // CHUNK) == c_i).astype(jnp.float32)
    hist = lax.dot_general(oh, a_mat, (((1,), (0,)), ((), ())),
                           preferred_element_type=jnp.float32)
    cc_r = lax.broadcasted_iota(jnp.int32, (NCHUNK, NCHUNK), 0)
    cc_c = lax.broadcasted_iota(jnp.int32, (NCHUNK, NCHUNK), 1)
    lc = (cc_r < cc_c).astype(jnp.float32)
    cumh = lax.dot_general(hist, lc, (((1,), (0,)), ((), ())),
                           preferred_element_type=jnp.float32)
    pad_off = (blk_start_i * BLK).astype(jnp.float32)
    base = cumh + pad_off

    u_r = lax.broadcasted_iota(jnp.int32, (CHUNK, CHUNK), 0)
    u_c = lax.broadcasted_iota(jnp.int32, (CHUNK, CHUNK), 1)
    t1 = (u_r < u_c).astype(jnp.float32)
    rows = []
    for c in range(NCHUNK):
        ohc = oh[:, CHUNK * c:CHUNK * (c + 1)]
        rank = lax.dot_general(ohc, t1, (((1,), (0,)), ((), ())),
                               preferred_element_type=jnp.float32)
        slot = jnp.sum(ohc * (base[:, c:c + 1] + rank),
                       axis=0, keepdims=True)
        rows.append(slot)
    dest_ref[...] = jnp.concatenate(rows, axis=0).astype(jnp.int32)


def _ffn_kernel(be_ref, x_ref, w1_any, b1_ref, w2_any, b2_ref, y_ref,
                stage1, stage2, w1b, w2b, sem1, sem2):
    i = pl.program_id(0)
    n_active = be_ref[NA_LANE]

    @pl.when(i < n_active)
    def _():
        e = be_ref[i]
        prev = be_ref[jnp.maximum(i - 1, 0)]
        changed = jnp.logical_or(i == 0, prev != e)

        @pl.when(i == 0)
        def _():
            pltpu.make_async_copy(w1_any.at[e], stage1, sem1).start()
            pltpu.make_async_copy(w2_any.at[e], stage2, sem2).start()

        @pl.when(changed)
        def _():
            pltpu.make_async_copy(w1_any.at[e], stage1, sem1).wait()
            pltpu.make_async_copy(w2_any.at[e], stage2, sem2).wait()
            w1b[...] = stage1[...].astype(jnp.bfloat16)
            w2b[...] = stage2[...].astype(jnp.bfloat16)

        xb = x_ref[...].astype(jnp.bfloat16)
        h = lax.dot_general(xb, w1b[...], (((1,), (1,)), ((), ())),
                            preferred_element_type=jnp.float32)
        h = h + b1_ref[0]
        h = 0.5 * h * (1.0 + lax.erf(h * _SQRT1_2))
        hb = h.astype(jnp.bfloat16)
        y = lax.dot_general(hb, w2b[...], (((1,), (1,)), ((), ())),
                            preferred_element_type=jnp.float32)
        y_ref[...] = y + b2_ref[0]

        nxt = be_ref[i + 1]

        @pl.when(jnp.logical_and(i + 1 < n_active, nxt != e))
        def _():
            pltpu.make_async_copy(w1_any.at[nxt], stage1, sem1).start()
            pltpu.make_async_copy(w2_any.at[nxt], stage2, sem2).start()


@functools.cache
def _sc_kernels():
    mesh = plsc.VectorSubcoreMesh(core_axis_name="c", subcore_axis_name="s")

    scratch = [
        pltpu.VMEM((NLEG, LEG), jnp.int32),
        pltpu.VMEM((LEG, D_MODEL), jnp.float32),
        pltpu.VMEM((LEG, D_MODEL), jnp.float32),
        pltpu.SemaphoreType.DMA,
        pltpu.SemaphoreType.DMA,
    ]

    @functools.partial(
        pl.kernel, mesh=mesh,
        out_type=jax.ShapeDtypeStruct((PADN, D_MODEL), jnp.float32),
        scratch_types=scratch,
    )
    def sc_scatter(x_hbm, dest_hbm, xs_hbm, didx_v, rows_a, rows_b, sem_a, sem_b):
        wid = lax.axis_index("s") * 2 + lax.axis_index("c")
        pltpu.sync_copy(dest_hbm.at[wid], didx_v)
        bufs = (rows_a, rows_b)
        sems = (sem_a, sem_b)
        pending = [None, None]
        for j in range(NLEG):
            b = j % 2
            if pending[b] is not None:
                pending[b].wait()
            base = wid * CHUNK + j * LEG
            pltpu.sync_copy(x_hbm.at[pl.ds(base, LEG)], bufs[b])
            pending[b] = pltpu.async_copy(
                bufs[b], xs_hbm.at[didx_v.at[j]], sems[b])
        for h in pending:
            h.wait()

    @functools.partial(
        pl.kernel, mesh=mesh,
        out_type=jax.ShapeDtypeStruct((N_TOK, D_MODEL), jnp.float32),
        scratch_types=scratch,
    )
    def sc_gather(ys_hbm, dest_hbm, out_hbm, didx_v, rows_a, rows_b, sem_a, sem_b):
        wid = lax.axis_index("s") * 2 + lax.axis_index("c")
        pltpu.sync_copy(dest_hbm.at[wid], didx_v)
        bufs = (rows_a, rows_b)
        sems = (sem_a, sem_b)
        pending = [None, None]
        for j in range(NLEG):
            b = j % 2
            if pending[b] is not None:
                pending[b].wait()
                pltpu.sync_copy(bufs[b], out_hbm.at[pl.ds(wid * CHUNK + (j - 2) * LEG, LEG)])
            pending[b] = pltpu.async_copy(
                ys_hbm.at[didx_v.at[j]], bufs[b], sems[b])
        for j, h in enumerate(pending):
            h.wait()
            pltpu.sync_copy(bufs[j], out_hbm.at[pl.ds(wid * CHUNK + (NLEG - 2 + j) * LEG, LEG)])

    return sc_scatter, sc_gather


def kernel(x, gate_w, gate_b, w1, b1, w2, b2):
    Bb, Ll, D = x.shape
    x_flat = x.reshape(N_TOK, D)

    dest, be, p_col, c_col, aux11 = pl.pallas_call(
        _gate_kernel,
        out_shape=[
            jax.ShapeDtypeStruct((NCHUNK, CHUNK), jnp.int32),
            jax.ShapeDtypeStruct((1, 128), jnp.int32),
            jax.ShapeDtypeStruct((N_EXPERT, 1), jnp.float32),
            jax.ShapeDtypeStruct((N_EXPERT, 1), jnp.float32),
            jax.ShapeDtypeStruct((1, 1), jnp.float32),
        ],
    )(gate_w, x_flat, gate_b.reshape(N_EXPERT, 1))

    sc_scatter, sc_gather = _sc_kernels()
    dest3 = dest.reshape(NCHUNK, NLEG, LEG)
    x_sorted = sc_scatter(x_flat, dest3)

    be_s = be.reshape(128)
    y_sorted = pl.pallas_call(
        _ffn_kernel,
        grid_spec=pltpu.PrefetchScalarGridSpec(
            num_scalar_prefetch=1,
            grid=(NB,),
            in_specs=[
                pl.BlockSpec((BLK, D_MODEL), lambda i, be: (i, 0)),
                pl.BlockSpec(memory_space=pl.ANY),
                pl.BlockSpec((1, 1, D_HIDDEN), lambda i, be: (be[i], 0, 0)),
                pl.BlockSpec(memory_space=pl.ANY),
                pl.BlockSpec((1, 1, D_MODEL), lambda i, be: (be[i], 0, 0)),
            ],
            out_specs=pl.BlockSpec((BLK, D_MODEL), lambda i, be: (i, 0)),
            scratch_shapes=[
                pltpu.VMEM((D_HIDDEN, D_MODEL), jnp.float32),
                pltpu.VMEM((D_MODEL, D_HIDDEN), jnp.float32),
                pltpu.VMEM((D_HIDDEN, D_MODEL), jnp.bfloat16),
                pltpu.VMEM((D_MODEL, D_HIDDEN), jnp.bfloat16),
                pltpu.SemaphoreType.DMA,
                pltpu.SemaphoreType.DMA,
            ],
        ),
        out_shape=jax.ShapeDtypeStruct((PADN, D_MODEL), jnp.float32),
        compiler_params=pltpu.CompilerParams(
            vmem_limit_bytes=62 * 1024 * 1024),
    )(be_s, x_sorted, w1, b1.reshape(N_EXPERT, 1, D_HIDDEN),
      w2, b2.reshape(N_EXPERT, 1, D_MODEL))

    out_flat = sc_gather(y_sorted, dest3)

    out = out_flat.reshape(Bb, Ll, D)
    return (out, aux11[0, 0], p_col[:, 0], c_col[:, 0])


# speedup vs baseline: 8.0762x; 1.0690x over previous
import functools

import jax
import jax.numpy as jnp
from jax import lax
from jax.experimental import pallas as pl
from jax.experimental.pallas import tpu as pltpu
from jax.experimental.pallas import tpu_sc as plsc

D_MODEL = 1024
D_HIDDEN = 4096
N_EXPERT = 8
AUX_COEF = 0.01
N_TOK = 4096
BLK = 256
NB = N_TOK // BLK + N_EXPERT
PADN = NB * BLK
NCHUNK = 32
CHUNK = N_TOK // NCHUNK
NLEG = 4
LEG = CHUNK // NLEG
NA_LANE = 120

_SQRT1_2 = 0.7071067811865476


def _gate_kernel(gw_ref, x_ref, gb_ref, dest_ref, be_ref, p_ref, c_ref, aux_ref):
    gw = gw_ref[...]
    x = x_ref[...]
    logits = lax.dot_general(gw, x, (((1,), (1,)), ((), ())),
                             preferred_element_type=jnp.float32)
    logits = logits + gb_ref[...]
    m = jnp.max(logits, axis=0, keepdims=True)
    ex = jnp.exp(logits - m)
    probs = ex / jnp.sum(ex, axis=0, keepdims=True)

    pmax = jnp.max(probs, axis=0, keepdims=True)
    eidx = lax.broadcasted_iota(jnp.int32, (N_EXPERT, N_TOK), 0)
    top1 = jnp.min(jnp.where(probs >= pmax, eidx, N_EXPERT),
                   axis=0, keepdims=True)
    oh = (eidx == top1).astype(jnp.float32)

    counts = jnp.sum(oh, axis=1, keepdims=True)
    p_vec = jnp.sum(probs, axis=1, keepdims=True) / N_TOK
    c_vec = counts / N_TOK
    p_ref[...] = p_vec
    c_ref[...] = c_vec
    aux_ref[...] = jnp.reshape(
        jnp.sum(p_vec * c_vec) * (N_EXPERT * AUX_COEF), (1, 1))

    cnt_i = counts.astype(jnp.int32)
    nb = (cnt_i + (BLK - 1)) // BLK
    e_r = lax.broadcasted_iota(jnp.int32, (N_EXPERT, N_EXPERT), 0)
    e_c = lax.broadcasted_iota(jnp.int32, (N_EXPERT, N_EXPERT), 1)
    l_strict = (e_c < e_r).astype(jnp.float32)
    blk_start = lax.dot_general(l_strict, nb.astype(jnp.float32),
                                (((1,), (0,)), ((), ())),
                                preferred_element_type=jnp.float32)
    blk_start_i = blk_start.astype(jnp.int32)

    bs_b = jnp.broadcast_to(blk_start_i, (N_EXPERT, 128))
    ib = lax.broadcasted_iota(jnp.int32, (N_EXPERT, 128), 1)
    be = jnp.sum((bs_b <= ib).astype(jnp.int32), axis=0, keepdims=True) - 1
    n_active = jnp.sum(nb)
    lane = lax.broadcasted_iota(jnp.int32, (1, 128), 1)
    be_ref[...] = jnp.where(lane == NA_LANE, n_active, be)

    t_i = lax.broadcasted_iota(jnp.int32, (N_TOK, NCHUNK), 0)
    c_i = lax.broadcasted_iota(jnp.int32, (N_TOK, NCHUNK), 1)
    a_mat = ((t_i // CHUNK) == c_i).astype(jnp.float32)
    hist = lax.dot_general(oh, a_mat, (((1,), (0,)), ((), ())),
                           preferred_element_type=jnp.float32)
    cc_r = lax.broadcasted_iota(jnp.int32, (NCHUNK, NCHUNK), 0)
    cc_c = lax.broadcasted_iota(jnp.int32, (NCHUNK, NCHUNK), 1)
    lc = (cc_r < cc_c).astype(jnp.float32)
    cumh = lax.dot_general(hist, lc, (((1,), (0,)), ((), ())),
                           preferred_element_type=jnp.float32)
    pad_off = (blk_start_i * BLK).astype(jnp.float32)
    base = cumh + pad_off

    u_r = lax.broadcasted_iota(jnp.int32, (CHUNK, CHUNK), 0)
    u_c = lax.broadcasted_iota(jnp.int32, (CHUNK, CHUNK), 1)
    t1 = (u_r < u_c).astype(jnp.float32)
    rows = []
    for c in range(NCHUNK):
        ohc = oh[:, CHUNK * c:CHUNK * (c + 1)]
        rank = lax.dot_general(ohc, t1, (((1,), (0,)), ((), ())),
                               preferred_element_type=jnp.float32)
        slot = jnp.sum(ohc * (base[:, c:c + 1] + rank),
                       axis=0, keepdims=True)
        rows.append(slot)
    dest_ref[...] = jnp.concatenate(rows, axis=0).astype(jnp.int32)


def _ffn_kernel(be_ref, x_ref, w1_any, b1_ref, w2_any, b2_ref, y_ref,
                stage1, stage2, w1b, w2b, sem1, sem2):
    i = pl.program_id(0)
    n_active = be_ref[NA_LANE]

    @pl.when(i < n_active)
    def _():
        e = be_ref[i]
        prev = be_ref[jnp.maximum(i - 1, 0)]
        changed = jnp.logical_or(i == 0, prev != e)

        @pl.when(i == 0)
        def _():
            pltpu.make_async_copy(w1_any.at[e], stage1, sem1).start()
            pltpu.make_async_copy(w2_any.at[e], stage2, sem2).start()

        @pl.when(changed)
        def _():
            pltpu.make_async_copy(w1_any.at[e], stage1, sem1).wait()
            w1b[...] = stage1[...].astype(jnp.bfloat16)

        xb = x_ref[...].astype(jnp.bfloat16)
        h = lax.dot_general(xb, w1b[...], (((1,), (1,)), ((), ())),
                            preferred_element_type=jnp.float32)
        h = h + b1_ref[0]
        h = 0.5 * h * (1.0 + lax.erf(h * _SQRT1_2))
        hb = h.astype(jnp.bfloat16)

        @pl.when(changed)
        def _():
            pltpu.make_async_copy(w2_any.at[e], stage2, sem2).wait()
            w2b[...] = stage2[...].astype(jnp.bfloat16)
        y = lax.dot_general(hb, w2b[...], (((1,), (1,)), ((), ())),
                            preferred_element_type=jnp.float32)
        y_ref[...] = y + b2_ref[0]

        nxt = be_ref[i + 1]

        @pl.when(jnp.logical_and(i + 1 < n_active, nxt != e))
        def _():
            pltpu.make_async_copy(w1_any.at[nxt], stage1, sem1).start()
            pltpu.make_async_copy(w2_any.at[nxt], stage2, sem2).start()


@functools.cache
def _sc_kernels():
    mesh = plsc.VectorSubcoreMesh(core_axis_name="c", subcore_axis_name="s")

    scratch = [
        pltpu.VMEM((NLEG, LEG), jnp.int32),
        pltpu.VMEM((LEG, D_MODEL), jnp.float32),
        pltpu.VMEM((LEG, D_MODEL), jnp.float32),
        pltpu.SemaphoreType.DMA,
        pltpu.SemaphoreType.DMA,
    ]

    @functools.partial(
        pl.kernel, mesh=mesh,
        out_type=jax.ShapeDtypeStruct((PADN, D_MODEL), jnp.float32),
        scratch_types=scratch,
    )
    def sc_scatter(x_hbm, dest_hbm, xs_hbm, didx_v, rows_a, rows_b, sem_a, sem_b):
        wid = lax.axis_index("s") * 2 + lax.axis_index("c")
        pltpu.sync_copy(dest_hbm.at[wid], didx_v)
        bufs = (rows_a, rows_b)
        sems = (sem_a, sem_b)
        pending = [None, None]
        for j in range(NLEG):
            b = j % 2
            if pending[b] is not None:
                pending[b].wait()
            base = wid * CHUNK + j * LEG
            pltpu.sync_copy(x_hbm.at[pl.ds(base, LEG)], bufs[b])
            pending[b] = pltpu.async_copy(
                bufs[b], xs_hbm.at[didx_v.at[j]], sems[b])
        for h in pending:
            h.wait()

    @functools.partial(
        pl.kernel, mesh=mesh,
        out_type=jax.ShapeDtypeStruct((N_TOK, D_MODEL), jnp.float32),
        scratch_types=scratch,
    )
    def sc_gather(ys_hbm, dest_hbm, out_hbm, didx_v, rows_a, rows_b, sem_a, sem_b):
        wid = lax.axis_index("s") * 2 + lax.axis_index("c")
        pltpu.sync_copy(dest_hbm.at[wid], didx_v)
        bufs = (rows_a, rows_b)
        sems = (sem_a, sem_b)
        pending = [None, None]
        for j in range(NLEG):
            b = j % 2
            if pending[b] is not None:
                pending[b].wait()
                pltpu.sync_copy(bufs[b], out_hbm.at[pl.ds(wid * CHUNK + (j - 2) * LEG, LEG)])
            pending[b] = pltpu.async_copy(
                ys_hbm.at[didx_v.at[j]], bufs[b], sems[b])
        for j, h in enumerate(pending):
            h.wait()
            pltpu.sync_copy(bufs[j], out_hbm.at[pl.ds(wid * CHUNK + (NLEG - 2 + j) * LEG, LEG)])

    return sc_scatter, sc_gather


def kernel(x, gate_w, gate_b, w1, b1, w2, b2):
    Bb, Ll, D = x.shape
    x_flat = x.reshape(N_TOK, D)

    dest, be, p_col, c_col, aux11 = pl.pallas_call(
        _gate_kernel,
        out_shape=[
            jax.ShapeDtypeStruct((NCHUNK, CHUNK), jnp.int32),
            jax.ShapeDtypeStruct((1, 128), jnp.int32),
            jax.ShapeDtypeStruct((N_EXPERT, 1), jnp.float32),
            jax.ShapeDtypeStruct((N_EXPERT, 1), jnp.float32),
            jax.ShapeDtypeStruct((1, 1), jnp.float32),
        ],
    )(gate_w, x_flat, gate_b.reshape(N_EXPERT, 1))

    sc_scatter, sc_gather = _sc_kernels()
    dest3 = dest.reshape(NCHUNK, NLEG, LEG)
    x_sorted = sc_scatter(x_flat, dest3)

    be_s = be.reshape(128)
    y_sorted = pl.pallas_call(
        _ffn_kernel,
        grid_spec=pltpu.PrefetchScalarGridSpec(
            num_scalar_prefetch=1,
            grid=(NB,),
            in_specs=[
                pl.BlockSpec((BLK, D_MODEL), lambda i, be: (i, 0)),
                pl.BlockSpec(memory_space=pl.ANY),
                pl.BlockSpec((1, 1, D_HIDDEN), lambda i, be: (be[i], 0, 0)),
                pl.BlockSpec(memory_space=pl.ANY),
                pl.BlockSpec((1, 1, D_MODEL), lambda i, be: (be[i], 0, 0)),
            ],
            out_specs=pl.BlockSpec((BLK, D_MODEL), lambda i, be: (i, 0)),
            scratch_shapes=[
                pltpu.VMEM((D_HIDDEN, D_MODEL), jnp.float32),
                pltpu.VMEM((D_MODEL, D_HIDDEN), jnp.float32),
                pltpu.VMEM((D_HIDDEN, D_MODEL), jnp.bfloat16),
                pltpu.VMEM((D_MODEL, D_HIDDEN), jnp.bfloat16),
                pltpu.SemaphoreType.DMA,
                pltpu.SemaphoreType.DMA,
            ],
        ),
        out_shape=jax.ShapeDtypeStruct((PADN, D_MODEL), jnp.float32),
        compiler_params=pltpu.CompilerParams(
            vmem_limit_bytes=62 * 1024 * 1024),
    )(be_s, x_sorted, w1, b1.reshape(N_EXPERT, 1, D_HIDDEN),
      w2, b2.reshape(N_EXPERT, 1, D_MODEL))

    out_flat = sc_gather(y_sorted, dest3)

    out = out_flat.reshape(Bb, Ll, D)
    return (out, aux11[0, 0], p_col[:, 0], c_col[:, 0])


# speedup vs baseline: 9.1444x; 1.1323x over previous
import functools

import jax
import jax.numpy as jnp
from jax import lax
from jax.experimental import pallas as pl
from jax.experimental.pallas import tpu as pltpu
from jax.experimental.pallas import tpu_sc as plsc

D_MODEL = 1024
D_HIDDEN = 4096
N_EXPERT = 8
AUX_COEF = 0.01
N_TOK = 4096
BLK = 256
NB = N_TOK // BLK + N_EXPERT
PADN = NB * BLK
NCHUNK = 32
CHUNK = N_TOK // NCHUNK
NLEG = 4
LEG = CHUNK // NLEG
NA_LANE = 120

_SQRT1_2 = 0.7071067811865476


def _gate_kernel(gw_ref, x_ref, gb_ref, dest_ref, be_ref, p_ref, c_ref, aux_ref):
    gw = gw_ref[...]
    x = x_ref[...]
    logits = lax.dot_general(gw, x, (((1,), (1,)), ((), ())),
                             preferred_element_type=jnp.float32)
    logits = logits + gb_ref[...]
    m = jnp.max(logits, axis=0, keepdims=True)
    ex = jnp.exp(logits - m)
    probs = ex / jnp.sum(ex, axis=0, keepdims=True)

    pmax = jnp.max(probs, axis=0, keepdims=True)
    eidx = lax.broadcasted_iota(jnp.int32, (N_EXPERT, N_TOK), 0)
    top1 = jnp.min(jnp.where(probs >= pmax, eidx, N_EXPERT),
                   axis=0, keepdims=True)
    oh = (eidx == top1).astype(jnp.float32)

    counts = jnp.sum(oh, axis=1, keepdims=True)
    p_vec = jnp.sum(probs, axis=1, keepdims=True) / N_TOK
    c_vec = counts / N_TOK
    p_ref[...] = p_vec
    c_ref[...] = c_vec
    aux_ref[...] = jnp.reshape(
        jnp.sum(p_vec * c_vec) * (N_EXPERT * AUX_COEF), (1, 1))

    cnt_i = counts.astype(jnp.int32)
    nb = (cnt_i + (BLK - 1)) // BLK
    e_r = lax.broadcasted_iota(jnp.int32, (N_EXPERT, N_EXPERT), 0)
    e_c = lax.broadcasted_iota(jnp.int32, (N_EXPERT, N_EXPERT), 1)
    l_strict = (e_c < e_r).astype(jnp.float32)
    blk_start = lax.dot_general(l_strict, nb.astype(jnp.float32),
                                (((1,), (0,)), ((), ())),
                                preferred_element_type=jnp.float32)
    blk_start_i = blk_start.astype(jnp.int32)

    bs_b = jnp.broadcast_to(blk_start_i, (N_EXPERT, 128))
    ib = lax.broadcasted_iota(jnp.int32, (N_EXPERT, 128), 1)
    be = jnp.sum((bs_b <= ib).astype(jnp.int32), axis=0, keepdims=True) - 1
    n_active = jnp.sum(nb)
    lane = lax.broadcasted_iota(jnp.int32, (1, 128), 1)
    be_row = jnp.where(lane == NA_LANE, n_active, be)

    nb_b = lax.dot_general(jnp.ones((N_EXPERT, 1), jnp.float32),
                           nb.astype(jnp.float32), (((1,), (1,)), ((), ())),
                           preferred_element_type=jnp.float32)
    cand = jnp.where((e_c > e_r) & (nb_b > 0.5), e_c, N_EXPERT)
    mn = jnp.min(cand, axis=1, keepdims=True)
    e_col = lax.broadcasted_iota(jnp.int32, (N_EXPERT, 1), 0)
    nxe = jnp.where(mn > N_EXPERT - 1, e_col, mn)
    cmp8 = (lax.broadcasted_iota(jnp.int32, (N_EXPERT, 128), 0)
            == jnp.broadcast_to(be, (N_EXPERT, 128)))
    ne_row = jnp.sum(jnp.where(cmp8, jnp.broadcast_to(nxe, (N_EXPERT, 128)), 0),
                     axis=0, keepdims=True)
    be_ref[...] = jnp.concatenate([be_row, ne_row], axis=0)

    t_i = lax.broadcasted_iota(jnp.int32, (N_TOK, NCHUNK), 0)
    c_i = lax.broadcasted_iota(jnp.int32, (N_TOK, NCHUNK), 1)
    a_mat = ((t_i // CHUNK) == c_i).astype(jnp.float32)
    hist = lax.dot_general(oh, a_mat, (((1,), (0,)), ((), ())),
                           preferred_element_type=jnp.float32)
    cc_r = lax.broadcasted_iota(jnp.int32, (NCHUNK, NCHUNK), 0)
    cc_c = lax.broadcasted_iota(jnp.int32, (NCHUNK, NCHUNK), 1)
    lc = (cc_r < cc_c).astype(jnp.float32)
    cumh = lax.dot_general(hist, lc, (((1,), (0,)), ((), ())),
                           preferred_element_type=jnp.float32)
    pad_off = (blk_start_i * BLK).astype(jnp.float32)
    base = cumh + pad_off

    u_r = lax.broadcasted_iota(jnp.int32, (CHUNK, CHUNK), 0)
    u_c = lax.broadcasted_iota(jnp.int32, (CHUNK, CHUNK), 1)
    t1 = (u_r < u_c).astype(jnp.float32)
    rows = []
    for c in range(NCHUNK):
        ohc = oh[:, CHUNK * c:CHUNK * (c + 1)]
        rank = lax.dot_general(ohc, t1, (((1,), (0,)), ((), ())),
                               preferred_element_type=jnp.float32)
        slot = jnp.sum(ohc * (base[:, c:c + 1] + rank),
                       axis=0, keepdims=True)
        rows.append(slot)
    dest_ref[...] = jnp.concatenate(rows, axis=0).astype(jnp.int32)


def _ffn_kernel(be_ref, x_ref, w1_any, b1_ref, w2_any, b2_ref, y_ref,
                stage1, stage2, w1b, w2b, sem1, sem2):
    i = pl.program_id(0)
    n_active = be_ref[0, NA_LANE]

    @pl.when(i < n_active)
    def _():
        e = be_ref[0, i]
        prev = be_ref[0, jnp.maximum(i - 1, 0)]
        changed = jnp.logical_or(i == 0, prev != e)
        ne = be_ref[1, i]
        last = jnp.logical_and(be_ref[0, i + 1] != e, ne != e)

        @pl.when(i == 0)
        def _():
            pltpu.make_async_copy(w1_any.at[e], stage1, sem1).start()
            pltpu.make_async_copy(w2_any.at[e], stage2, sem2).start()
            pltpu.make_async_copy(w1_any.at[e], stage1, sem1).wait()
            w1b[...] = stage1[...].astype(jnp.bfloat16)
            pltpu.make_async_copy(w2_any.at[e], stage2, sem2).wait()
            w2b[...] = stage2[...].astype(jnp.bfloat16)

        @pl.when(jnp.logical_and(changed, ne != e))
        def _():
            pltpu.make_async_copy(w1_any.at[ne], stage1, sem1).start()
            pltpu.make_async_copy(w2_any.at[ne], stage2, sem2).start()

        xb = x_ref[...].astype(jnp.bfloat16)
        h = lax.dot_general(xb, w1b[...], (((1,), (1,)), ((), ())),
                            preferred_element_type=jnp.float32)

        @pl.when(last)
        def _():
            pltpu.make_async_copy(w1_any.at[ne], stage1, sem1).wait()
            w1b[...] = stage1[...].astype(jnp.bfloat16)

        h = h + b1_ref[0]
        h = 0.5 * h * (1.0 + lax.erf(h * _SQRT1_2))
        hb = h.astype(jnp.bfloat16)
        y = lax.dot_general(hb, w2b[...], (((1,), (1,)), ((), ())),
                            preferred_element_type=jnp.float32)
        y_ref[...] = y + b2_ref[0]

        @pl.when(last)
        def _():
            pltpu.make_async_copy(w2_any.at[ne], stage2, sem2).wait()
            w2b[...] = stage2[...].astype(jnp.bfloat16)


@functools.cache
def _sc_kernels():
    mesh = plsc.VectorSubcoreMesh(core_axis_name="c", subcore_axis_name="s")

    scratch = [
        pltpu.VMEM((NLEG, LEG), jnp.int32),
        pltpu.VMEM((LEG, D_MODEL), jnp.float32),
        pltpu.VMEM((LEG, D_MODEL), jnp.float32),
        pltpu.SemaphoreType.DMA,
        pltpu.SemaphoreType.DMA,
    ]

    @functools.partial(
        pl.kernel, mesh=mesh,
        out_type=jax.ShapeDtypeStruct((PADN, D_MODEL), jnp.float32),
        scratch_types=scratch,
    )
    def sc_scatter(x_hbm, dest_hbm, xs_hbm, didx_v, rows_a, rows_b, sem_a, sem_b):
        wid = lax.axis_index("s") * 2 + lax.axis_index("c")
        pltpu.sync_copy(dest_hbm.at[wid], didx_v)
        bufs = (rows_a, rows_b)
        sems = (sem_a, sem_b)
        pending = [None, None]
        for j in range(NLEG):
            b = j % 2
            if pending[b] is not None:
                pending[b].wait()
            base = wid * CHUNK + j * LEG
            pltpu.sync_copy(x_hbm.at[pl.ds(base, LEG)], bufs[b])
            pending[b] = pltpu.async_copy(
                bufs[b], xs_hbm.at[didx_v.at[j]], sems[b])
        for h in pending:
            h.wait()

    @functools.partial(
        pl.kernel, mesh=mesh,
        out_type=jax.ShapeDtypeStruct((N_TOK, D_MODEL), jnp.float32),
        scratch_types=scratch,
    )
    def sc_gather(ys_hbm, dest_hbm, out_hbm, didx_v, rows_a, rows_b, sem_a, sem_b):
        wid = lax.axis_index("s") * 2 + lax.axis_index("c")
        pltpu.sync_copy(dest_hbm.at[wid], didx_v)
        bufs = (rows_a, rows_b)
        sems = (sem_a, sem_b)
        pending = [None, None]
        for j in range(NLEG):
            b = j % 2
            if pending[b] is not None:
                pending[b].wait()
                pltpu.sync_copy(bufs[b], out_hbm.at[pl.ds(wid * CHUNK + (j - 2) * LEG, LEG)])
            pending[b] = pltpu.async_copy(
                ys_hbm.at[didx_v.at[j]], bufs[b], sems[b])
        for j, h in enumerate(pending):
            h.wait()
            pltpu.sync_copy(bufs[j], out_hbm.at[pl.ds(wid * CHUNK + (NLEG - 2 + j) * LEG, LEG)])

    return sc_scatter, sc_gather


def kernel(x, gate_w, gate_b, w1, b1, w2, b2):
    Bb, Ll, D = x.shape
    x_flat = x.reshape(N_TOK, D)

    dest, be, p_col, c_col, aux11 = pl.pallas_call(
        _gate_kernel,
        out_shape=[
            jax.ShapeDtypeStruct((NCHUNK, CHUNK), jnp.int32),
            jax.ShapeDtypeStruct((2, 128), jnp.int32),
            jax.ShapeDtypeStruct((N_EXPERT, 1), jnp.float32),
            jax.ShapeDtypeStruct((N_EXPERT, 1), jnp.float32),
            jax.ShapeDtypeStruct((1, 1), jnp.float32),
        ],
    )(gate_w, x_flat, gate_b.reshape(N_EXPERT, 1))

    sc_scatter, sc_gather = _sc_kernels()
    dest3 = dest.reshape(NCHUNK, NLEG, LEG)
    x_sorted = sc_scatter(x_flat, dest3)

    y_sorted = pl.pallas_call(
        _ffn_kernel,
        grid_spec=pltpu.PrefetchScalarGridSpec(
            num_scalar_prefetch=1,
            grid=(NB,),
            in_specs=[
                pl.BlockSpec((BLK, D_MODEL), lambda i, be: (i, 0)),
                pl.BlockSpec(memory_space=pl.ANY),
                pl.BlockSpec((1, 1, D_HIDDEN), lambda i, be: (be[0, i], 0, 0)),
                pl.BlockSpec(memory_space=pl.ANY),
                pl.BlockSpec((1, 1, D_MODEL), lambda i, be: (be[0, i], 0, 0)),
            ],
            out_specs=pl.BlockSpec((BLK, D_MODEL), lambda i, be: (i, 0)),
            scratch_shapes=[
                pltpu.VMEM((D_HIDDEN, D_MODEL), jnp.float32),
                pltpu.VMEM((D_MODEL, D_HIDDEN), jnp.float32),
                pltpu.VMEM((D_HIDDEN, D_MODEL), jnp.bfloat16),
                pltpu.VMEM((D_MODEL, D_HIDDEN), jnp.bfloat16),
                pltpu.SemaphoreType.DMA,
                pltpu.SemaphoreType.DMA,
            ],
        ),
        out_shape=jax.ShapeDtypeStruct((PADN, D_MODEL), jnp.float32),
        compiler_params=pltpu.CompilerParams(
            vmem_limit_bytes=62 * 1024 * 1024),
    )(be, x_sorted, w1, b1.reshape(N_EXPERT, 1, D_HIDDEN),
      w2, b2.reshape(N_EXPERT, 1, D_MODEL))

    out_flat = sc_gather(y_sorted, dest3)

    out = out_flat.reshape(Bb, Ll, D)
    return (out, aux11[0, 0], p_col[:, 0], c_col[:, 0])


# speedup vs baseline: 9.1819x; 1.0041x over previous
import functools

import jax
import jax.numpy as jnp
from jax import lax
from jax.experimental import pallas as pl
from jax.experimental.pallas import tpu as pltpu
from jax.experimental.pallas import tpu_sc as plsc

D_MODEL = 1024
D_HIDDEN = 4096
N_EXPERT = 8
AUX_COEF = 0.01
N_TOK = 4096
BLK = 256
NB = N_TOK // BLK + N_EXPERT
PADN = NB * BLK
NCHUNK = 32
CHUNK = N_TOK // NCHUNK
NLEG = 4
LEG = CHUNK // NLEG
NA_LANE = 120

_SQRT1_2 = 0.7071067811865476


def _gate_kernel(gw_ref, x_ref, gb_ref, dest_ref, be_ref, p_ref, c_ref, aux_ref):
    gw = gw_ref[...]
    x = x_ref[...]
    logits = lax.dot_general(gw, x, (((1,), (1,)), ((), ())),
                             preferred_element_type=jnp.float32)
    logits = logits + gb_ref[...]
    m = jnp.max(logits, axis=0, keepdims=True)
    ex = jnp.exp(logits - m)
    probs = ex / jnp.sum(ex, axis=0, keepdims=True)

    pmax = jnp.max(probs, axis=0, keepdims=True)
    eidx = lax.broadcasted_iota(jnp.int32, (N_EXPERT, N_TOK), 0)
    top1 = jnp.min(jnp.where(probs >= pmax, eidx, N_EXPERT),
                   axis=0, keepdims=True)
    oh = (eidx == top1).astype(jnp.float32)

    counts = jnp.sum(oh, axis=1, keepdims=True)
    p_vec = jnp.sum(probs, axis=1, keepdims=True) / N_TOK
    c_vec = counts / N_TOK
    p_ref[...] = p_vec
    c_ref[...] = c_vec
    aux_ref[...] = jnp.reshape(
        jnp.sum(p_vec * c_vec) * (N_EXPERT * AUX_COEF), (1, 1))

    cnt_i = counts.astype(jnp.int32)
    nb = (cnt_i + (BLK - 1)) // BLK
    e_r = lax.broadcasted_iota(jnp.int32, (N_EXPERT, N_EXPERT), 0)
    e_c = lax.broadcasted_iota(jnp.int32, (N_EXPERT, N_EXPERT), 1)
    l_strict = (e_c < e_r).astype(jnp.float32)
    blk_start = lax.dot_general(l_strict, nb.astype(jnp.float32),
                                (((1,), (0,)), ((), ())),
                                preferred_element_type=jnp.float32)
    blk_start_i = blk_start.astype(jnp.int32)

    bs_b = jnp.broadcast_to(blk_start_i, (N_EXPERT, 128))
    ib = lax.broadcasted_iota(jnp.int32, (N_EXPERT, 128), 1)
    be = jnp.sum((bs_b <= ib).astype(jnp.int32), axis=0, keepdims=True) - 1
    n_active = jnp.sum(nb)
    lane = lax.broadcasted_iota(jnp.int32, (1, 128), 1)
    be_row = jnp.where(lane == NA_LANE, n_active, be)

    nb_b = lax.dot_general(jnp.ones((N_EXPERT, 1), jnp.float32),
                           nb.astype(jnp.float32), (((1,), (1,)), ((), ())),
                           preferred_element_type=jnp.float32)
    cand = jnp.where((e_c > e_r) & (nb_b > 0.5), e_c, N_EXPERT)
    mn = jnp.min(cand, axis=1, keepdims=True)
    e_col = lax.broadcasted_iota(jnp.int32, (N_EXPERT, 1), 0)
    nxe = jnp.where(mn > N_EXPERT - 1, e_col, mn)
    e_iota8 = lax.broadcasted_iota(jnp.int32, (N_EXPERT, 128), 0)
    be_b = jnp.broadcast_to(be, (N_EXPERT, 128))
    cmp8 = e_iota8 == be_b
    ne_row = jnp.sum(jnp.where(cmp8, jnp.broadcast_to(nxe, (N_EXPERT, 128)), 0),
                     axis=0, keepdims=True)
    nonempty_b = jnp.broadcast_to(nb > 0, (N_EXPERT, 128))
    seg_idx = jnp.sum(((e_iota8 < be_b) & nonempty_b).astype(jnp.int32),
                      axis=0, keepdims=True)
    par_row = jnp.bitwise_and(seg_idx, 1)
    bs_bb = jnp.broadcast_to(blk_start_i, (N_EXPERT, 128))
    pos_row = lane - jnp.sum(jnp.where(cmp8, bs_bb, 0), axis=0, keepdims=True)
    be_ref[...] = jnp.concatenate([be_row, ne_row, par_row, pos_row], axis=0)

    t_i = lax.broadcasted_iota(jnp.int32, (N_TOK, NCHUNK), 0)
    c_i = lax.broadcasted_iota(jnp.int32, (N_TOK, NCHUNK), 1)
    a_mat = ((t_i // CHUNK) == c_i).astype(jnp.float32)
    hist = lax.dot_general(oh, a_mat, (((1,), (0,)), ((), ())),
                           preferred_element_type=jnp.float32)
    cc_r = lax.broadcasted_iota(jnp.int32, (NCHUNK, NCHUNK), 0)
    cc_c = lax.broadcasted_iota(jnp.int32, (NCHUNK, NCHUNK), 1)
    lc = (cc_r < cc_c).astype(jnp.float32)
    cumh = lax.dot_general(hist, lc, (((1,), (0,)), ((), ())),
                           preferred_element_type=jnp.float32)
    pad_off = (blk_start_i * BLK).astype(jnp.float32)
    base = cumh + pad_off

    u_r = lax.broadcasted_iota(jnp.int32, (CHUNK, CHUNK), 0)
    u_c = lax.broadcasted_iota(jnp.int32, (CHUNK, CHUNK), 1)
    t1 = (u_r < u_c).astype(jnp.float32)
    rows = []
    for c in range(NCHUNK):
        ohc = oh[:, CHUNK * c:CHUNK * (c + 1)]
        rank = lax.dot_general(ohc, t1, (((1,), (0,)), ((), ())),
                               preferred_element_type=jnp.float32)
        slot = jnp.sum(ohc * (base[:, c:c + 1] + rank),
                       axis=0, keepdims=True)
        rows.append(slot)
    dest_ref[...] = jnp.concatenate(rows, axis=0).astype(jnp.int32)


HHALF = D_HIDDEN // 2
DHALF = D_MODEL // 2


def _ffn_kernel(be_ref, x_ref, w1_any, b1_ref, w2_any, b2_ref, y_ref,
                stage_a, stage_b, w1b2, w2b2, sem_a, sem_b):
    i = pl.program_id(0)
    n_active = be_ref[0, NA_LANE]

    @pl.when(i < n_active)
    def _():
        e = be_ref[0, i]
        ne = be_ref[1, i]
        par = be_ref[2, i]
        pos = be_ref[3, i]
        q = 1 - par
        valid = ne != e
        last = jnp.logical_and(be_ref[0, i + 1] != e, valid)

        def w1_copy(ex, half):
            return pltpu.make_async_copy(
                w1_any.at[ex, pl.ds(half * HHALF, HHALF)], stage_a, sem_a)

        def w2_copy(ex, half):
            return pltpu.make_async_copy(
                w2_any.at[ex, pl.ds(half * DHALF, DHALF)], stage_b, sem_b)

        def conv_w1(dst, half):
            w1b2[dst, pl.ds(half * HHALF, HHALF), :] = (
                stage_a[...].astype(jnp.bfloat16))

        def conv_w2(dst, half):
            w2b2[dst, pl.ds(half * DHALF, DHALF), :] = (
                stage_b[...].astype(jnp.bfloat16))

        @pl.when(i == 0)
        def _():
            w1_copy(e, 0).start()
            w2_copy(e, 0).start()
            w1_copy(e, 0).wait()
            conv_w1(par, 0)
            w1_copy(e, 1).start()
            w2_copy(e, 0).wait()
            conv_w2(par, 0)
            w2_copy(e, 1).start()
            w1_copy(e, 1).wait()
            conv_w1(par, 1)
            w2_copy(e, 1).wait()
            conv_w2(par, 1)

        xb = x_ref[...].astype(jnp.bfloat16)
        h = lax.dot_general(xb, w1b2[par], (((1,), (1,)), ((), ())),
                            preferred_element_type=jnp.float32)
        h = h + b1_ref[0]
        h = 0.5 * h * (1.0 + lax.erf(h * _SQRT1_2))
        hb = h.astype(jnp.bfloat16)
        y = lax.dot_general(hb, w2b2[par], (((1,), (1,)), ((), ())),
                            preferred_element_type=jnp.float32)
        y_ref[...] = y + b2_ref[0]

        @pl.when(jnp.logical_and(valid, pos == 0))
        def _():
            w1_copy(ne, 0).start()
            w2_copy(ne, 0).start()

        step1 = jnp.logical_or(pos == 1, jnp.logical_and(last, pos < 1))

        @pl.when(jnp.logical_and(valid, step1))
        def _():
            w1_copy(ne, 0).wait()
            conv_w1(q, 0)
            w1_copy(ne, 1).start()
            w2_copy(ne, 0).wait()
            conv_w2(q, 0)
            w2_copy(ne, 1).start()

        step2 = jnp.logical_or(pos == 2, jnp.logical_and(last, pos < 2))

        @pl.when(jnp.logical_and(valid, step2))
        def _():
            w1_copy(ne, 1).wait()
            conv_w1(q, 1)
            w2_copy(ne, 1).wait()
            conv_w2(q, 1)


@functools.cache
def _sc_kernels():
    mesh = plsc.VectorSubcoreMesh(core_axis_name="c", subcore_axis_name="s")

    scratch = [
        pltpu.VMEM((NLEG, LEG), jnp.int32),
        pltpu.VMEM((LEG, D_MODEL), jnp.float32),
        pltpu.VMEM((LEG, D_MODEL), jnp.float32),
        pltpu.SemaphoreType.DMA,
        pltpu.SemaphoreType.DMA,
    ]

    @functools.partial(
        pl.kernel, mesh=mesh,
        out_type=jax.ShapeDtypeStruct((PADN, D_MODEL), jnp.float32),
        scratch_types=scratch,
    )
    def sc_scatter(x_hbm, dest_hbm, xs_hbm, didx_v, rows_a, rows_b, sem_a, sem_b):
        wid = lax.axis_index("s") * 2 + lax.axis_index("c")
        pltpu.sync_copy(dest_hbm.at[wid], didx_v)
        bufs = (rows_a, rows_b)
        sems = (sem_a, sem_b)
        pending = [None, None]
        for j in range(NLEG):
            b = j % 2
            if pending[b] is not None:
                pending[b].wait()
            base = wid * CHUNK + j * LEG
            pltpu.sync_copy(x_hbm.at[pl.ds(base, LEG)], bufs[b])
            pending[b] = pltpu.async_copy(
                bufs[b], xs_hbm.at[didx_v.at[j]], sems[b])
        for h in pending:
            h.wait()

    @functools.partial(
        pl.kernel, mesh=mesh,
        out_type=jax.ShapeDtypeStruct((N_TOK, D_MODEL), jnp.float32),
        scratch_types=scratch,
    )
    def sc_gather(ys_hbm, dest_hbm, out_hbm, didx_v, rows_a, rows_b, sem_a, sem_b):
        wid = lax.axis_index("s") * 2 + lax.axis_index("c")
        pltpu.sync_copy(dest_hbm.at[wid], didx_v)
        bufs = (rows_a, rows_b)
        sems = (sem_a, sem_b)
        pending = [None, None]
        for j in range(NLEG):
            b = j % 2
            if pending[b] is not None:
                pending[b].wait()
                pltpu.sync_copy(bufs[b], out_hbm.at[pl.ds(wid * CHUNK + (j - 2) * LEG, LEG)])
            pending[b] = pltpu.async_copy(
                ys_hbm.at[didx_v.at[j]], bufs[b], sems[b])
        for j, h in enumerate(pending):
            h.wait()
            pltpu.sync_copy(bufs[j], out_hbm.at[pl.ds(wid * CHUNK + (NLEG - 2 + j) * LEG, LEG)])

    return sc_scatter, sc_gather


def kernel(x, gate_w, gate_b, w1, b1, w2, b2):
    Bb, Ll, D = x.shape
    x_flat = x.reshape(N_TOK, D)

    dest, be, p_col, c_col, aux11 = pl.pallas_call(
        _gate_kernel,
        out_shape=[
            jax.ShapeDtypeStruct((NCHUNK, CHUNK), jnp.int32),
            jax.ShapeDtypeStruct((4, 128), jnp.int32),
            jax.ShapeDtypeStruct((N_EXPERT, 1), jnp.float32),
            jax.ShapeDtypeStruct((N_EXPERT, 1), jnp.float32),
            jax.ShapeDtypeStruct((1, 1), jnp.float32),
        ],
    )(gate_w, x_flat, gate_b.reshape(N_EXPERT, 1))

    sc_scatter, sc_gather = _sc_kernels()
    dest3 = dest.reshape(NCHUNK, NLEG, LEG)
    x_sorted = sc_scatter(x_flat, dest3)

    y_sorted = pl.pallas_call(
        _ffn_kernel,
        grid_spec=pltpu.PrefetchScalarGridSpec(
            num_scalar_prefetch=1,
            grid=(NB,),
            in_specs=[
                pl.BlockSpec((BLK, D_MODEL), lambda i, be: (i, 0)),
                pl.BlockSpec(memory_space=pl.ANY),
                pl.BlockSpec((1, 1, D_HIDDEN), lambda i, be: (be[0, i], 0, 0)),
                pl.BlockSpec(memory_space=pl.ANY),
                pl.BlockSpec((1, 1, D_MODEL), lambda i, be: (be[0, i], 0, 0)),
            ],
            out_specs=pl.BlockSpec((BLK, D_MODEL), lambda i, be: (i, 0)),
            scratch_shapes=[
                pltpu.VMEM((D_HIDDEN // 2, D_MODEL), jnp.float32),
                pltpu.VMEM((D_MODEL // 2, D_HIDDEN), jnp.float32),
                pltpu.VMEM((2, D_HIDDEN, D_MODEL), jnp.bfloat16),
                pltpu.VMEM((2, D_MODEL, D_HIDDEN), jnp.bfloat16),
                pltpu.SemaphoreType.DMA,
                pltpu.SemaphoreType.DMA,
            ],
        ),
        out_shape=jax.ShapeDtypeStruct((PADN, D_MODEL), jnp.float32),
        compiler_params=pltpu.CompilerParams(
            vmem_limit_bytes=62 * 1024 * 1024),
    )(be, x_sorted, w1, b1.reshape(N_EXPERT, 1, D_HIDDEN),
      w2, b2.reshape(N_EXPERT, 1, D_MODEL))

    out_flat = sc_gather(y_sorted, dest3)

    out = out_flat.reshape(Bb, Ll, D)
    return (out, aux11[0, 0], p_col[:, 0], c_col[:, 0])
